# baseline (speedup 1.0000x reference)
import functools

import jax
import jax.numpy as jnp
from jax import lax
from jax.experimental import pallas as pl
from jax.experimental.pallas import tpu as pltpu
from jax.experimental.pallas import tpu_sc as plsc

_N = 50000
_D = 512
_S = _N - 1
_TR = 80
_NT = _N // _TR
_NW = 32
_TPW = 20
_PAD = _N + 16
_NEG = float("-inf")

_mesh = plsc.VectorSubcoreMesh(core_axis_name="c", subcore_axis_name="s")


def _k1_body(hv_hbm, w_hbm, scores_hbm,
             w_v, hb0, hb1, sc_all, sem0, sem1, semo):
    wid = lax.axis_index("s") * 2 + lax.axis_index("c")
    iota = lax.iota(jnp.int32, 16)
    lane0 = iota == 0
    pltpu.sync_copy(w_hbm.at[0, pl.ds(0, _D)], w_v)
    wv = [w_v[pl.ds(16 * k, 16)] for k in range(32)]
    hbufs = (hb0, hb1)
    sems = (sem0, sem1)

    def tile_id(l):
        t = wid + _NW * l
        return jnp.where(t < _NT, t, 0)

    def in_copy(l):
        t = tile_id(l)
        return pltpu.make_async_copy(
            hv_hbm.at[pl.ds(t * _TR, _TR)], hbufs[l % 2], sems[l % 2])

    def out_copy(l):
        t = tile_id(l)
        return pltpu.make_async_copy(
            sc_all.at[pl.ds(l * _TR, _TR)],
            scores_hbm.at[pl.ds(t * _TR, _TR)], semo)

    def in_copy_d(l, b):
        t = tile_id(l)
        return pltpu.make_async_copy(
            hv_hbm.at[pl.ds(t * _TR, _TR)], hbufs[b], sems[b])

    in_copy_d(0, 0).start()
    in_copy_d(1, 1).start()

    def pair_body(l0, carry):
        for b in (0, 1):
            l = 2 * l0 + b
            in_copy_d(l, b).wait()
            hb = hbufs[b]
            base = l * _TR

            @plsc.parallel_loop(0, _TR, 1, unroll=8)
            def _row(rr, hb=hb, base=base):
                ps = [hb[rr, pl.ds(16 * k, 16)] * wv[k] for k in range(32)]
                while len(ps) > 1:
                    ps = [ps[i] + ps[i + 1] for i in range(0, len(ps), 2)]
                plsc.store_scatter(
                    sc_all, [jnp.full((16,), base + rr, jnp.int32)],
                    jnp.full((16,), jnp.sum(ps[0])), mask=lane0)

            out_copy(l).start()

            @pl.when(l + 2 < _TPW)
            def _():
                in_copy_d(l + 2, b).start()
        return carry

    lax.fori_loop(0, _TPW // 2, pair_body, 0)

    def drain(l, carry):
        out_copy(l).wait()
        return carry

    lax.fori_loop(0, _TPW, drain, 0)


_k1 = functools.partial(
    pl.kernel,
    out_type=[jax.ShapeDtypeStruct((_PAD,), jnp.float32)],
    mesh=_mesh,
    compiler_params=pltpu.CompilerParams(needs_layout_passes=False),
    scratch_types=[
        pltpu.VMEM((_D,), jnp.float32),
        pltpu.VMEM((_TR, _D), jnp.float32),
        pltpu.VMEM((_TR, _D), jnp.float32),
        pltpu.VMEM((_TPW * _TR,), jnp.float32),
        pltpu.SemaphoreType.DMA,
        pltpu.SemaphoreType.DMA,
        pltpu.SemaphoreType.DMA,
    ],
)(_k1_body)


def _ep_body(dest_ref, sc_ref, probs_ref, logp_ref):
    s = sc_ref[...]
    col = lax.broadcasted_iota(jnp.int32, (1, _PAD), 1)
    s = jnp.where(col >= _S, _NEG, s)
    m = jnp.max(s)
    e = jnp.exp(s - m)
    z = jnp.sum(e)
    probs_ref[...] = e[:, :_S] * (1.0 / z)
    d = dest_ref[0]
    sd = jnp.max(jnp.where(col == d, s, _NEG))
    logp_ref[...] = jnp.broadcast_to(sd - m - jnp.log(z), (1, 1))


def kernel(hv, W, b, dest):
    del b
    dest_arr = jnp.asarray(dest, dtype=jnp.int32).reshape((1,))
    (scores,) = _k1(hv, W)
    probs, logp = pl.pallas_call(
        _ep_body,
        in_specs=[
            pl.BlockSpec(memory_space=pltpu.SMEM),
            pl.BlockSpec((1, _PAD), lambda: (0, 0)),
        ],
        out_specs=[
            pl.BlockSpec((1, _S), lambda: (0, 0)),
            pl.BlockSpec((1, 1), lambda: (0, 0)),
        ],
        out_shape=[
            jax.ShapeDtypeStruct((1, _S), jnp.float32),
            jax.ShapeDtypeStruct((1, 1), jnp.float32),
        ],
    )(dest_arr, scores.reshape(1, _PAD))
    return (probs, logp)


# speedup vs baseline: 1.6452x; 1.6452x over previous
import functools

import jax
import jax.numpy as jnp
from jax import lax
from jax.experimental import pallas as pl
from jax.experimental.pallas import tpu as pltpu
from jax.experimental.pallas import tpu_sc as plsc

_N = 50000
_D = 512
_S = _N - 1
_RTC = 35840
_BRT = 5120
_NBT = _RTC // _BRT
_TR = 80
_NT = (_N - _RTC) // _TR
_NW = 32
_TPW = -(-_NT // _NW)
_PADSC = _N - _RTC + 16
_NEG = float("-inf")

_mesh = plsc.VectorSubcoreMesh(core_axis_name="c", subcore_axis_name="s")


def _k1_body(hv_hbm, w_hbm, scores_hbm,
             w_v, hb0, hb1, sc_all, sem0, sem1, semo):
    wid = lax.axis_index("s") * 2 + lax.axis_index("c")
    iota = lax.iota(jnp.int32, 16)
    lane0 = iota == 0
    pltpu.sync_copy(w_hbm.at[0, pl.ds(0, _D)], w_v)
    wv = [w_v[pl.ds(16 * k, 16)] for k in range(32)]
    hbufs = (hb0, hb1)
    sems = (sem0, sem1)

    def tile_id(l):
        t = wid + _NW * l
        return jnp.where(t < _NT, t, 0)

    def in_copy(l):
        t = tile_id(l)
        return pltpu.make_async_copy(
            hv_hbm.at[pl.ds(_RTC + t * _TR, _TR)], hbufs[l % 2], sems[l % 2])

    def out_copy(l):
        t = tile_id(l)
        return pltpu.make_async_copy(
            sc_all.at[pl.ds(l * _TR, _TR)],
            scores_hbm.at[pl.ds(t * _TR, _TR)], semo)

    in_copy(0).start()

    for l in range(_TPW):
        if l + 1 < _TPW:
            in_copy(l + 1).start()
        in_copy(l).wait()
        hb = hbufs[l % 2]
        base = l * _TR

        @plsc.parallel_loop(0, _TR, 1, unroll=8)
        def _row(rr, hb=hb, base=base):
            ps = [hb[rr, pl.ds(16 * k, 16)] * wv[k] for k in range(32)]
            while len(ps) > 1:
                ps = [ps[i] + ps[i + 1] for i in range(0, len(ps), 2)]
            plsc.store_scatter(
                sc_all, [jnp.full((16,), base + rr, jnp.int32)],
                jnp.full((16,), jnp.sum(ps[0])), mask=lane0)

        out_copy(l).start()

    for l in range(_TPW):
        out_copy(l).wait()


_k1 = functools.partial(
    pl.kernel,
    out_type=[jax.ShapeDtypeStruct((_PADSC,), jnp.float32)],
    mesh=_mesh,
    compiler_params=pltpu.CompilerParams(needs_layout_passes=False),
    scratch_types=[
        pltpu.VMEM((_D,), jnp.float32),
        pltpu.VMEM((_TR, _D), jnp.float32),
        pltpu.VMEM((_TR, _D), jnp.float32),
        pltpu.VMEM((_TPW * _TR,), jnp.float32),
        pltpu.SemaphoreType.DMA,
        pltpu.SemaphoreType.DMA,
        pltpu.SemaphoreType.DMA,
    ],
)(_k1_body)


def _tc_body(hv_ref, w_ref, out_ref):
    w1 = w_ref[:, :_D]
    out_ref[...] = jax.lax.dot_general(
        w1, hv_ref[...], (((1,), (1,)), ((), ())),
        preferred_element_type=jnp.float32)


def _ep_body(dest_ref, tcs_ref, scs_ref, probs_ref, logp_ref):
    s1 = tcs_ref[...]
    s2 = scs_ref[...]
    c1 = lax.broadcasted_iota(jnp.int32, (1, _RTC), 1)
    c2 = lax.broadcasted_iota(jnp.int32, (1, _PADSC), 1) + _RTC
    s2 = jnp.where(c2 >= _S, _NEG, s2)
    m = jnp.maximum(jnp.max(s1), jnp.max(s2))
    e1 = jnp.exp(s1 - m)
    e2 = jnp.exp(s2 - m)
    rz = 1.0 / (jnp.sum(e1) + jnp.sum(e2))
    probs_ref[:, :_RTC] = e1 * rz
    probs_ref[:, _RTC:] = (e2 * rz)[:, :_S - _RTC]
    d = dest_ref[0]
    sd = jnp.maximum(jnp.max(jnp.where(c1 == d, s1, _NEG)),
                     jnp.max(jnp.where(c2 == d, s2, _NEG)))
    logp_ref[...] = jnp.broadcast_to(sd - m + jnp.log(rz), (1, 1))


def kernel(hv, W, b, dest):
    del b
    dest_arr = jnp.asarray(dest, dtype=jnp.int32).reshape((1,))
    (sc_scores,) = _k1(hv, W)
    tc_scores = pl.pallas_call(
        _tc_body,
        grid=(_NBT,),
        in_specs=[
            pl.BlockSpec((_BRT, _D), lambda i: (i, 0)),
            pl.BlockSpec((1, 2 * _D), lambda i: (0, 0)),
        ],
        out_specs=pl.BlockSpec((1, _BRT), lambda i: (0, i)),
        out_shape=jax.ShapeDtypeStruct((1, _RTC), jnp.float32),
    )(hv, W)
    probs, logp = pl.pallas_call(
        _ep_body,
        in_specs=[
            pl.BlockSpec(memory_space=pltpu.SMEM),
            pl.BlockSpec((1, _RTC), lambda: (0, 0)),
            pl.BlockSpec((1, _PADSC), lambda: (0, 0)),
        ],
        out_specs=[
            pl.BlockSpec((1, _S), lambda: (0, 0)),
            pl.BlockSpec((1, 1), lambda: (0, 0)),
        ],
        out_shape=[
            jax.ShapeDtypeStruct((1, _S), jnp.float32),
            jax.ShapeDtypeStruct((1, 1), jnp.float32),
        ],
    )(dest_arr, tc_scores, sc_scores.reshape(1, _PADSC))
    return (probs, logp)


# speedup vs baseline: 1.9213x; 1.1678x over previous
import functools

import jax
import jax.numpy as jnp
from jax import lax
from jax.experimental import pallas as pl
from jax.experimental.pallas import tpu as pltpu
from jax.experimental.pallas import tpu_sc as plsc

_N = 50000
_D = 512
_S = _N - 1
_RTC = 38400
_BRT = 3840
_NBT = _RTC // _BRT
_TR = 80
_NT = (_N - _RTC) // _TR
_NW = 32
_TPW = -(-_NT // _NW)
_PADSC = _N - _RTC + 16
_NEG = float("-inf")

_mesh = plsc.VectorSubcoreMesh(core_axis_name="c", subcore_axis_name="s")


def _k1_body(hv_hbm, w_hbm, scores_hbm,
             w_v, hb0, hb1, sc_all, sem0, sem1, semo):
    wid = lax.axis_index("s") * 2 + lax.axis_index("c")
    iota = lax.iota(jnp.int32, 16)
    lane0 = iota == 0
    pltpu.sync_copy(w_hbm.at[0, pl.ds(0, _D)], w_v)
    wv = [w_v[pl.ds(16 * k, 16)] for k in range(32)]
    hbufs = (hb0, hb1)
    sems = (sem0, sem1)

    def tile_id(l):
        t = wid + _NW * l
        return jnp.where(t < _NT, t, 0)

    def in_copy(l):
        t = tile_id(l)
        return pltpu.make_async_copy(
            hv_hbm.at[pl.ds(_RTC + t * _TR, _TR)], hbufs[l % 2], sems[l % 2])

    def out_copy(l):
        t = tile_id(l)
        return pltpu.make_async_copy(
            sc_all.at[pl.ds(l * _TR, _TR)],
            scores_hbm.at[pl.ds(t * _TR, _TR)], semo)

    in_copy(0).start()

    for l in range(_TPW):
        if l + 1 < _TPW:
            in_copy(l + 1).start()
        in_copy(l).wait()
        hb = hbufs[l % 2]
        base = l * _TR

        @plsc.parallel_loop(0, _TR, 1, unroll=4)
        def _row(rr, hb=hb, base=base):
            ps = [hb[rr, pl.ds(16 * k, 16)] * wv[k] for k in range(32)]
            while len(ps) > 1:
                ps = [ps[i] + ps[i + 1] for i in range(0, len(ps), 2)]
            plsc.store_scatter(
                sc_all, [jnp.full((16,), base + rr, jnp.int32)],
                jnp.full((16,), jnp.sum(ps[0])), mask=lane0)

        out_copy(l).start()

    for l in range(_TPW):
        out_copy(l).wait()


_k1 = functools.partial(
    pl.kernel,
    out_type=[jax.ShapeDtypeStruct((_PADSC,), jnp.float32)],
    mesh=_mesh,
    compiler_params=pltpu.CompilerParams(needs_layout_passes=False),
    scratch_types=[
        pltpu.VMEM((_D,), jnp.float32),
        pltpu.VMEM((_TR, _D), jnp.float32),
        pltpu.VMEM((_TR, _D), jnp.float32),
        pltpu.VMEM((_TPW * _TR,), jnp.float32),
        pltpu.SemaphoreType.DMA,
        pltpu.SemaphoreType.DMA,
        pltpu.SemaphoreType.DMA,
    ],
)(_k1_body)


def _tc_body(hv_ref, w_ref, out_ref):
    w1 = w_ref[:, :_D]
    out_ref[...] = jax.lax.dot_general(
        w1, hv_ref[...], (((1,), (1,)), ((), ())),
        preferred_element_type=jnp.float32)


def _ep_body(dest_ref, tcs_ref, scs_ref, probs_ref, logp_ref):
    s1 = tcs_ref[...]
    s2 = scs_ref[...]
    c1 = lax.broadcasted_iota(jnp.int32, (1, _RTC), 1)
    c2 = lax.broadcasted_iota(jnp.int32, (1, _PADSC), 1) + _RTC
    s2 = jnp.where(c2 >= _S, _NEG, s2)
    m = jnp.maximum(jnp.max(s1), jnp.max(s2))
    e1 = jnp.exp(s1 - m)
    e2 = jnp.exp(s2 - m)
    rz = 1.0 / (jnp.sum(e1) + jnp.sum(e2))
    probs_ref[:, :_RTC] = e1 * rz
    probs_ref[:, _RTC:] = (e2 * rz)[:, :_S - _RTC]
    d = dest_ref[0]
    sd = jnp.maximum(jnp.max(jnp.where(c1 == d, s1, _NEG)),
                     jnp.max(jnp.where(c2 == d, s2, _NEG)))
    logp_ref[...] = jnp.broadcast_to(sd - m + jnp.log(rz), (1, 1))


def kernel(hv, W, b, dest):
    del b
    dest_arr = jnp.asarray(dest, dtype=jnp.int32).reshape((1,))
    (sc_scores,) = _k1(hv, W)
    tc_scores = pl.pallas_call(
        _tc_body,
        grid=(_NBT,),
        in_specs=[
            pl.BlockSpec((_BRT, _D), lambda i: (i, 0)),
            pl.BlockSpec((1, 2 * _D), lambda i: (0, 0)),
        ],
        out_specs=pl.BlockSpec((1, _BRT), lambda i: (0, i)),
        out_shape=jax.ShapeDtypeStruct((1, _RTC), jnp.float32),
    )(hv, W)
    probs, logp = pl.pallas_call(
        _ep_body,
        in_specs=[
            pl.BlockSpec(memory_space=pltpu.SMEM),
            pl.BlockSpec((1, _RTC), lambda: (0, 0)),
            pl.BlockSpec((1, _PADSC), lambda: (0, 0)),
        ],
        out_specs=[
            pl.BlockSpec((1, _S), lambda: (0, 0)),
            pl.BlockSpec((1, 1), lambda: (0, 0)),
        ],
        out_shape=[
            jax.ShapeDtypeStruct((1, _S), jnp.float32),
            jax.ShapeDtypeStruct((1, 1), jnp.float32),
        ],
    )(dest_arr, tc_scores, sc_scores.reshape(1, _PADSC))
    return (probs, logp)


# speedup vs baseline: 1.9717x; 1.0263x over previous
import functools

import jax
import jax.numpy as jnp
from jax import lax
from jax.experimental import pallas as pl
from jax.experimental.pallas import tpu as pltpu
from jax.experimental.pallas import tpu_sc as plsc

_N = 50000
_D = 512
_S = _N - 1
_RTC = 38400
_BRT = 3840
_NBT = _RTC // _BRT
_TR = 80
_NT = (_N - _RTC) // _TR
_NW = 32
_TPW = -(-_NT // _NW)
_PADSC = _N - _RTC + 16
_NEG = float("-inf")

_mesh = plsc.VectorSubcoreMesh(core_axis_name="c", subcore_axis_name="s")


def _k1_body(hv_hbm, w_hbm, scores_hbm,
             w_v, hb0, hb1, sc_all, sem0, sem1, semo):
    wid = lax.axis_index("s") * 2 + lax.axis_index("c")
    iota = lax.iota(jnp.int32, 16)
    lane0 = iota == 0
    pltpu.sync_copy(w_hbm.at[0, pl.ds(0, _D)], w_v)
    wv = [w_v[pl.ds(16 * k, 16)] for k in range(32)]
    hbufs = (hb0, hb1)
    sems = (sem0, sem1)

    def tile_id(l):
        t = wid + _NW * l
        return jnp.where(t < _NT, t, 0)

    def in_copy(l):
        t = tile_id(l)
        return pltpu.make_async_copy(
            hv_hbm.at[pl.ds(_RTC + t * _TR, _TR)], hbufs[l % 2], sems[l % 2])

    def out_copy(l):
        t = tile_id(l)
        return pltpu.make_async_copy(
            sc_all.at[pl.ds(l * _TR, _TR)],
            scores_hbm.at[pl.ds(t * _TR, _TR)], semo)

    in_copy(0).start()

    for l in range(_TPW):
        if l + 1 < _TPW:
            in_copy(l + 1).start()
        in_copy(l).wait()
        hb = hbufs[l % 2]
        base = l * _TR

        @plsc.parallel_loop(0, _TR, 1, unroll=4)
        def _row(rr, hb=hb, base=base):
            ps = [hb[rr, pl.ds(16 * k, 16)] * wv[k] for k in range(32)]
            while len(ps) > 1:
                ps = [ps[i] + ps[i + 1] for i in range(0, len(ps), 2)]
            plsc.store_scatter(
                sc_all, [jnp.full((16,), base + rr, jnp.int32)],
                jnp.full((16,), jnp.sum(ps[0])), mask=lane0)

        out_copy(l).start()

    for l in range(_TPW):
        out_copy(l).wait()


_k1 = functools.partial(
    pl.kernel,
    out_type=[jax.ShapeDtypeStruct((_PADSC,), jnp.float32)],
    mesh=_mesh,
    compiler_params=pltpu.CompilerParams(needs_layout_passes=False),
    scratch_types=[
        pltpu.VMEM((_D,), jnp.float32),
        pltpu.VMEM((_TR, _D), jnp.float32),
        pltpu.VMEM((_TR, _D), jnp.float32),
        pltpu.VMEM((_TPW * _TR,), jnp.float32),
        pltpu.SemaphoreType.DMA,
        pltpu.SemaphoreType.DMA,
        pltpu.SemaphoreType.DMA,
    ],
)(_k1_body)


def _tc_body(hv_ref, w_ref, out_ref):
    w1 = w_ref[:, :_D]
    out_ref[...] = jax.lax.dot_general(
        w1, hv_ref[...], (((1,), (1,)), ((), ())),
        preferred_element_type=jnp.float32)


def _ep_body(dest_ref, tcs_ref, scs_ref, probs_ref, logp_ref):
    s1 = tcs_ref[...]
    s2 = scs_ref[...].reshape(1, _PADSC)
    c1 = lax.broadcasted_iota(jnp.int32, (1, _RTC), 1)
    c2 = lax.broadcasted_iota(jnp.int32, (1, _PADSC), 1) + _RTC
    s2 = jnp.where(c2 >= _S, _NEG, s2)
    m = jnp.maximum(jnp.max(s1), jnp.max(s2))
    e1 = jnp.exp(s1 - m)
    e2 = jnp.exp(s2 - m)
    rz = 1.0 / (jnp.sum(e1) + jnp.sum(e2))
    probs_ref[:, :_RTC] = e1 * rz
    probs_ref[:, _RTC:] = (e2 * rz)[:, :_S - _RTC]
    d = dest_ref[0]
    sd = jnp.maximum(jnp.max(jnp.where(c1 == d, s1, _NEG)),
                     jnp.max(jnp.where(c2 == d, s2, _NEG)))
    logp_ref[...] = jnp.broadcast_to(sd - m + jnp.log(rz), (1, 1))


def kernel(hv, W, b, dest):
    del b
    dest_arr = jnp.asarray(dest, dtype=jnp.int32).reshape((1,))
    (sc_scores,) = _k1(hv, W)
    tc_scores = pl.pallas_call(
        _tc_body,
        grid=(_NBT,),
        in_specs=[
            pl.BlockSpec((_BRT, _D), lambda i: (i, 0)),
            pl.BlockSpec((1, 2 * _D), lambda i: (0, 0)),
        ],
        out_specs=pl.BlockSpec((1, _BRT), lambda i: (0, i)),
        out_shape=jax.ShapeDtypeStruct((1, _RTC), jnp.float32),
    )(hv, W)
    probs, logp = pl.pallas_call(
        _ep_body,
        in_specs=[
            pl.BlockSpec(memory_space=pltpu.SMEM),
            pl.BlockSpec((1, _RTC), lambda: (0, 0)),
            pl.BlockSpec((_PADSC,), lambda: (0,)),
        ],
        out_specs=[
            pl.BlockSpec((1, _S), lambda: (0, 0)),
            pl.BlockSpec((1, 1), lambda: (0, 0)),
        ],
        out_shape=[
            jax.ShapeDtypeStruct((1, _S), jnp.float32),
            jax.ShapeDtypeStruct((1, 1), jnp.float32),
        ],
    )(dest_arr, tc_scores, sc_scores)
    return (probs, logp)
